# Initial kernel scaffold; baseline (speedup 1.0000x reference)
import jax
import jax.numpy as jnp
from jax.experimental import pallas as pl


def kernel(x):
    raise NotImplementedError("write your pallas kernel here")


# speedup vs baseline: 3.0532x; 3.0532x over previous
import jax
import jax.numpy as jnp
import numpy as np
from jax.experimental import pallas as pl
from jax.experimental.pallas import tpu as pltpu

H = 128
W = 128
B = 16
WIN = 39
NC = WIN - B + 1
SIGMA = 0.1
LAMB3D = 2.7
K1 = 32
K2 = 64
SPD1 = 3
SPD2 = 4
KAISER_BETA = 2.0
NP1 = H - B
NPAD = 128
THR = LAMB3D * SIGMA


def _dct_mat_np(N):
    n = np.arange(N)
    M = np.cos(np.pi * (2.0 * n[None, :] + 1.0) * n[:, None] / (2.0 * N))
    M[0, :] = M[0, :] / np.sqrt(2.0)
    M = M * np.sqrt(2.0 / N)
    return M.astype(np.float32)


_DB = _dct_mat_np(B)
_DK1 = _dct_mat_np(K1)
_DK2 = _dct_mat_np(K2)
_kw = np.kaiser(B, KAISER_BETA)
_KAISER_FLAT = np.outer(_kw, _kw).astype(np.float32).reshape(1, B * B)

_MKRON = np.kron(_DB, _DB).astype(np.float32)

_EBIG = np.zeros((B * B, B * B), dtype=np.float32)
for _c in range(B):
    for _a in range(B):
        for _d in range(B):
            _EBIG[_c * B + _a, _a * B + _d] = _DB[_d, _c]


def _predct_body(img_ref, db_ref, eb_ref, out_ref):
    i = pl.program_id(0)
    rows = img_ref[pl.ds(i, B), :]
    R = jnp.dot(db_ref[...], rows, preferred_element_type=jnp.float32)
    Rt = R.T
    gbig = jnp.concatenate(
        [Rt[c:c + NP1, :] for c in range(B)], axis=1)
    out = jnp.concatenate(
        [jnp.dot(gbig, eb_ref[...], preferred_element_type=jnp.float32),
         jnp.zeros((NPAD - NP1, B * B), jnp.float32)], axis=0)
    out_ref[0] = out


def _predct(img):
    return pl.pallas_call(
        _predct_body,
        grid=(NP1,),
        in_specs=[
            pl.BlockSpec((H, W), lambda i: (0, 0)),
            pl.BlockSpec((B, B), lambda i: (0, 0)),
            pl.BlockSpec((B * B, B * B), lambda i: (0, 0)),
        ],
        out_specs=pl.BlockSpec((1, NPAD, B * B), lambda i: (i, 0, 0)),
        out_shape=jax.ShapeDtypeStruct((NP1, NPAD, B * B), jnp.float32),
    )(img, jnp.asarray(_DB), jnp.asarray(_EBIG))


def _gather_row(dct_ref, gr, gc):
    g8 = (gc // 8) * 8
    blk = dct_ref[pl.ds(gr, 1), pl.ds(g8, 8), :].reshape(8, B * B)
    oh = (jax.lax.broadcasted_iota(jnp.int32, (1, 8), 1)
          == gc - g8).astype(jnp.float32)
    return jnp.dot(oh, blk, preferred_element_type=jnp.float32)


def _select_gather(dct_ref, gather_refs, idxr_ref, idxc_ref, pr, pc, K):
    wr = jnp.clip(pr - (WIN // 2 - B // 2), 0, NP1 - NC)
    wc = jnp.clip(pc - (WIN // 2 - B // 2), 0, NP1 - NC)
    wc8 = (wc // 8) * 8
    rc = wc - wc8
    cand = dct_ref[pl.ds(wr, NC), pl.ds(wc8, 32), :]
    ref = _gather_row(dct_ref, pr, pc).reshape(1, 1, B * B)
    diff = cand - ref
    dist = jnp.sum(diff * diff, axis=2)
    ci = jax.lax.broadcasted_iota(jnp.int32, (NC, 32), 1)
    ri = jax.lax.broadcasted_iota(jnp.int32, (NC, 32), 0)
    valid = jnp.logical_and(ci >= rc, ci < rc + NC)
    tiekey = ri * NC + (ci - rc)
    dist = jnp.where(valid, dist, jnp.float32(jnp.inf))
    kiota = jax.lax.broadcasted_iota(jnp.int32, (K, B * B), 0)

    def body(k, carry):
        dist = carry[0]
        grps = carry[1:]
        m = jnp.min(dist)
        idx = jnp.min(jnp.where(dist == m, tiekey, jnp.int32(1 << 30)))
        di = idx // NC
        dj = idx - di * NC
        gr = wr + di
        gc = wc + dj
        idxr_ref[k] = gr
        idxc_ref[k] = gc
        new_grps = []
        for src_ref, grp in zip(gather_refs, grps):
            row = _gather_row(src_ref, gr, gc)
            new_grps.append(jnp.where(kiota == k, row, grp))
        dist = jnp.where(tiekey == idx, jnp.float32(jnp.inf), dist)
        return (dist,) + tuple(new_grps)

    init = (dist,) + tuple(
        jnp.zeros((K, B * B), jnp.float32) for _ in gather_refs)
    out = jax.lax.fori_loop(0, K, body, init)
    return list(out[1:])


def _flat_to_tile(row):
    tile = jnp.concatenate(
        [row[:, b * B:(b + 1) * B] for b in range(B)], axis=0)
    return jnp.concatenate(
        [tile, jnp.zeros((B, W - B), jnp.float32)], axis=1)


def _scatter(num_ref, den_ref, vals, idxr_ref, idxc_ref, wpad, K):
    zpad = jnp.zeros((8, W), jnp.float32)
    for k in range(K):
        gr = idxr_ref[k]
        gc = idxc_ref[k]
        r8 = (gr // 8) * 8
        rel = gr - r8
        vtile = pltpu.roll(_flat_to_tile(vals[k:k + 1, :]), gc, 1)
        wtile = pltpu.roll(wpad, gc, 1)
        v24 = pltpu.roll(jnp.concatenate([vtile, zpad], axis=0), rel, 0)
        w24 = pltpu.roll(jnp.concatenate([wtile, zpad], axis=0), rel, 0)
        num_ref[pl.ds(r8, NC), :] += v24
        den_ref[pl.ds(r8, NC), :] += w24


def _step1_body(dct_ref, dk_ref, mk_ref, kais_ref, num_ref, den_ref,
                idxr_ref, idxc_ref):
    i = pl.program_id(0)
    j = pl.program_id(1)

    @pl.when(jnp.logical_and(i == 0, j == 0))
    def _():
        num_ref[...] = jnp.zeros_like(num_ref)
        den_ref[...] = jnp.zeros_like(den_ref)

    pr = jnp.minimum(SPD1 * i, NP1 - 1)
    pc = jnp.minimum(SPD1 * j, NP1 - 1)
    (grp,) = _select_gather(dct_ref, [dct_ref], idxr_ref, idxc_ref,
                            pr, pc, K1)
    t = jnp.dot(dk_ref[...], grp, preferred_element_type=jnp.float32)
    t = jnp.where(jnp.abs(t) < THR, 0.0, t)
    nz = jnp.sum((t != 0.0).astype(jnp.float32))
    g = jnp.dot(dk_ref[...], t, preferred_element_type=jnp.float32)
    blocks = jnp.dot(g, mk_ref[...], preferred_element_type=jnp.float32)
    kais = kais_ref[...]
    w = jnp.where(nz < 1.0, kais, kais / (SIGMA ** 2 * nz))
    wpad = _flat_to_tile(w)
    _scatter(num_ref, den_ref, w * blocks, idxr_ref, idxc_ref, wpad, K1)


def _step2_body(dctb_ref, dctn_ref, dk_ref, mk_ref, kais_ref,
                num_ref, den_ref, idxr_ref, idxc_ref):
    i = pl.program_id(0)
    j = pl.program_id(1)

    @pl.when(jnp.logical_and(i == 0, j == 0))
    def _():
        num_ref[...] = jnp.zeros_like(num_ref)
        den_ref[...] = jnp.zeros_like(den_ref)

    pr = jnp.minimum(SPD2 * i, NP1 - 1)
    pc = jnp.minimum(SPD2 * j, NP1 - 1)
    gb, gn = _select_gather(dctb_ref, [dctb_ref, dctn_ref],
                            idxr_ref, idxc_ref, pr, pc, K2)
    dk = dk_ref[...]
    tb = jnp.dot(dk, gb, preferred_element_type=jnp.float32)
    wien = tb * tb / (tb * tb + SIGMA ** 2)
    tn = jnp.dot(dk, gn, preferred_element_type=jnp.float32)
    tf = wien * tn
    g = jnp.dot(dk, tf, preferred_element_type=jnp.float32)
    blocks = jnp.dot(g, mk_ref[...], preferred_element_type=jnp.float32)
    wnorm = jnp.sum(wien * wien)
    kais = kais_ref[...]
    w = jnp.where(wnorm < 1e-12, kais, kais / (SIGMA ** 2 * wnorm))
    wpad = _flat_to_tile(w)
    _scatter(num_ref, den_ref, w * blocks, idxr_ref, idxc_ref, wpad, K2)


def _grid_n(spd):
    return int((H - B) / spd) + 2


_TABLE_SPEC = pl.BlockSpec((NP1, NPAD, B * B), lambda i, j: (0, 0, 0))
_OUT_SPECS = [
    pl.BlockSpec((H, W), lambda i, j: (0, 0)),
    pl.BlockSpec((H, W), lambda i, j: (0, 0)),
]
_OUT_SHAPE = [
    jax.ShapeDtypeStruct((H, W), jnp.float32),
    jax.ShapeDtypeStruct((H, W), jnp.float32),
]


def _step1(dct_all):
    n = _grid_n(SPD1)
    return pl.pallas_call(
        _step1_body,
        grid=(n, n),
        in_specs=[
            _TABLE_SPEC,
            pl.BlockSpec((K1, K1), lambda i, j: (0, 0)),
            pl.BlockSpec((B * B, B * B), lambda i, j: (0, 0)),
            pl.BlockSpec((1, B * B), lambda i, j: (0, 0)),
        ],
        out_specs=_OUT_SPECS,
        out_shape=_OUT_SHAPE,
        scratch_shapes=[
            pltpu.SMEM((K1,), jnp.int32),
            pltpu.SMEM((K1,), jnp.int32),
        ],
    )(dct_all, jnp.asarray(_DK1), jnp.asarray(_MKRON),
      jnp.asarray(_KAISER_FLAT))


def _step2(dct_basic, dct_noisy):
    n = _grid_n(SPD2)
    return pl.pallas_call(
        _step2_body,
        grid=(n, n),
        in_specs=[
            _TABLE_SPEC,
            _TABLE_SPEC,
            pl.BlockSpec((K2, K2), lambda i, j: (0, 0)),
            pl.BlockSpec((B * B, B * B), lambda i, j: (0, 0)),
            pl.BlockSpec((1, B * B), lambda i, j: (0, 0)),
        ],
        out_specs=_OUT_SPECS,
        out_shape=_OUT_SHAPE,
        scratch_shapes=[
            pltpu.SMEM((K2,), jnp.int32),
            pltpu.SMEM((K2,), jnp.int32),
        ],
    )(dct_basic, dct_noisy, jnp.asarray(_DK2), jnp.asarray(_MKRON),
      jnp.asarray(_KAISER_FLAT))


@jax.jit
def kernel(x):
    img = x.reshape(H, W).astype(jnp.float32)
    dct_noisy = _predct(img)
    num1, den1 = _step1(dct_noisy)
    basic = num1 / jnp.where(den1 == 0.0, 1.0, den1)
    dct_basic = _predct(basic)
    num2, den2 = _step2(dct_basic, dct_noisy)
    final = num2 / jnp.where(den2 == 0.0, 1.0, den2)
    return final[None, None]
